# baseline (speedup 1.0000x reference)
import functools

import jax
import jax.numpy as jnp
from jax import lax
from jax.experimental import pallas as pl
from jax.experimental.pallas import tpu as pltpu
from jax.experimental.pallas import tpu_sc as plsc

_NC = 2
_NS = 16
_KG = 80
_KS = 40
_BE = 2000
_BN = 2000


def _make_gather(n, e, d):
    nw = _NC * _NS
    epw = e // nw
    grp = 80
    ngrp = epw // grp
    nsub = grp // _KG
    stage_rows = (n // (8 * _NS)) * 8
    stage_rem = n - stage_rows * _NS
    mesh = plsc.VectorSubcoreMesh(core_axis_name="c", subcore_axis_name="s")

    @functools.partial(
        pl.kernel,
        mesh=mesh,
        out_type=[
            jax.ShapeDtypeStruct((e, d), jnp.float32),
            jax.ShapeDtypeStruct((e, d), jnp.float32),
        ],
        scratch_types=[
            pltpu.VMEM((epw,), jnp.int32),
            pltpu.VMEM((epw,), jnp.int32),
            pltpu.VMEM((grp, d), jnp.float32),
            pltpu.VMEM((grp, d), jnp.float32),
            pltpu.VMEM_SHARED((n, d), jnp.float32),
            pltpu.SemaphoreType.DMA,
            pltpu.SemaphoreType.DMA,
            pltpu.SemaphoreType.DMA,
        ],
    )
    def gather_k(x_hbm, src_hbm, dst_hbm, out_src, out_dst,
                 si_v, di_v, b0, b1, x_sh, gsem, ws0, ws1):
        wid = lax.axis_index("s") * _NC + lax.axis_index("c")
        s = lax.axis_index("s")
        base0 = wid * epw
        xr0 = s * stage_rows
        pltpu.sync_copy(x_hbm.at[pl.ds(xr0, stage_rows)],
                        x_sh.at[pl.ds(xr0, stage_rows)])
        if stage_rem:
            @pl.when(s == 0)
            def _():
                pltpu.sync_copy(
                    x_hbm.at[pl.ds(stage_rows * _NS, stage_rem)],
                    x_sh.at[pl.ds(stage_rows * _NS, stage_rem)])
        pltpu.sync_copy(src_hbm.at[pl.ds(base0, epw)], si_v)
        pltpu.sync_copy(dst_hbm.at[pl.ds(base0, epw)], di_v)
        plsc.subcore_barrier()

        def group(g, carry):
            o = g * grp

            @pl.when(g > 0)
            def _():
                pltpu.make_async_copy(
                    b0, out_src.at[pl.ds(base0, grp)], ws0).wait()

            cps = [pltpu.async_copy(
                x_sh.at[si_v.at[pl.ds(o + j * _KG, _KG)]],
                b0.at[pl.ds(j * _KG, _KG)], gsem) for j in range(nsub)]
            for cp in cps:
                cp.wait()
            pltpu.async_copy(b0, out_src.at[pl.ds(base0 + o, grp)], ws0)

            @pl.when(g > 0)
            def _():
                pltpu.make_async_copy(
                    b1, out_dst.at[pl.ds(base0, grp)], ws1).wait()

            cps = [pltpu.async_copy(
                x_sh.at[di_v.at[pl.ds(o + j * _KG, _KG)]],
                b1.at[pl.ds(j * _KG, _KG)], gsem) for j in range(nsub)]
            for cp in cps:
                cp.wait()
            pltpu.async_copy(b1, out_dst.at[pl.ds(base0 + o, grp)], ws1)
            return carry

        lax.fori_loop(0, ngrp, group, 0)
        pltpu.make_async_copy(b0, out_src.at[pl.ds(base0, grp)], ws0).wait()
        pltpu.make_async_copy(b1, out_dst.at[pl.ds(base0, grp)], ws1).wait()

    return gather_k


def _make_scatter(n_pad, e, d):
    epc = e // _NC
    ept = epc // _NS
    grp = _KS
    ngrp = ept // grp
    ring = 5
    niter = ngrp // ring
    nrow = ept // _KS
    rows_per_tile = n_pad // _NS
    mesh = plsc.VectorSubcoreMesh(core_axis_name="c", subcore_axis_name="s")

    @functools.partial(
        pl.kernel,
        mesh=mesh,
        out_type=jax.ShapeDtypeStruct((_NC, n_pad, d), jnp.float32),
        scratch_types=[
            pltpu.VMEM((nrow, 1, _KS), jnp.int32),
        ] + [pltpu.VMEM((grp, d), jnp.float32) for _ in range(ring)]
        + [pltpu.VMEM_SHARED((n_pad, d), jnp.float32)]
        + [pltpu.SemaphoreType.DMA for _ in range(ring)]
        + [pltpu.SemaphoreType.DMA],
    )
    def scatter_k(msg_hbm, dst2_hbm, zeros_hbm, out_hbm, idx2_v, *rest):
        m = rest[:ring]
        agg_sh = rest[ring]
        ls = rest[ring + 1:2 * ring + 1]
        asem = rest[2 * ring + 1]
        c = lax.axis_index("c")
        s = lax.axis_index("s")
        r0 = s * rows_per_tile
        pltpu.sync_copy(zeros_hbm.at[pl.ds(r0, rows_per_tile)],
                        agg_sh.at[pl.ds(r0, rows_per_tile)])
        base0 = c * epc + s * ept
        row0 = base0 // _KS
        pltpu.sync_copy(dst2_hbm.at[pl.ds(row0, nrow)], idx2_v)
        plsc.subcore_barrier()
        for j in range(ring - 1):
            pltpu.async_copy(msg_hbm.at[pl.ds(base0 + j * grp, grp)],
                             m[j], ls[j])

        def drain_one_add():
            pltpu.make_async_copy(
                m[0], agg_sh.at[idx2_v.at[0, 0]], asem).wait()

        def body(p, carry):
            for j in range(ring):
                g = p * ring + j
                pltpu.make_async_copy(
                    msg_hbm.at[pl.ds(base0, grp)], m[j], ls[j]).wait()
                pltpu.async_copy(m[j], agg_sh.at[idx2_v.at[g, 0]],
                                 asem, add=True)

                @pl.when(g + ring - 1 < ngrp)
                def _():
                    drain_one_add()
                    jf = (j + ring - 1) % ring
                    pltpu.async_copy(
                        msg_hbm.at[pl.ds(base0 + (g + ring - 1) * grp, grp)],
                        m[jf], ls[jf])
            return carry

        lax.fori_loop(0, niter, body, 0)
        for _ in range(ring - 1):
            drain_one_add()
        plsc.subcore_barrier()
        pltpu.sync_copy(agg_sh.at[pl.ds(r0, rows_per_tile)],
                        out_hbm.at[c, pl.ds(r0, rows_per_tile)])

    return scatter_k


def _edge_body(xd_ref, xs_ref, attn_ref, wt_ref, g_ref, bt_ref, bb_ref,
               gate_ref, msg_ref):
    xd = xd_ref[...]
    xs = xs_ref[...]
    d = xd.shape[1]
    two_d = 2.0 * d
    ssum = jnp.sum(xd, axis=1, keepdims=True) + jnp.sum(xs, axis=1, keepdims=True)
    ssq = (jnp.sum(xd * xd, axis=1, keepdims=True)
           + jnp.sum(xs * xs, axis=1, keepdims=True))
    mu = ssum / two_d
    var = ssq / two_d - mu * mu
    rstd = lax.rsqrt(var + 1e-5)
    gamma = g_ref[...]
    beta = bt_ref[...]
    ln_d = (xd - mu) * rstd * gamma[:, :d] + beta[:, :d]
    ln_s = (xs - mu) * rstd * gamma[:, d:] + beta[:, d:]
    rd = jnp.maximum(ln_d, 0.0).astype(jnp.bfloat16)
    rs = jnp.maximum(ln_s, 0.0).astype(jnp.bfloat16)
    wt = wt_ref[...]
    h = (jnp.dot(rd, wt[:d], preferred_element_type=jnp.float32)
         + jnp.dot(rs, wt[d:], preferred_element_type=jnp.float32)
         + bb_ref[...])
    gate = jnp.mean(jax.nn.sigmoid(h), axis=1)
    scale = gate * attn_ref[0, 0, :]
    msg_ref[...] = xs * scale[:, None]
    gate_ref[0, 0, :] = gate


def _edge_compute(xg_dst, xg_src, attn3, wt, gamma2, beta2, bias2):
    e, d = xg_src.shape
    nb = e // _BE
    fd = wt.shape[1]
    grid = (nb,)
    gate3, msg = pl.pallas_call(
        _edge_body,
        grid=grid,
        in_specs=[
            pl.BlockSpec((_BE, d), lambda i: (i, 0)),
            pl.BlockSpec((_BE, d), lambda i: (i, 0)),
            pl.BlockSpec((1, 1, _BE), lambda i: (i, 0, 0)),
            pl.BlockSpec((2 * d, fd), lambda i: (0, 0)),
            pl.BlockSpec((1, 2 * d), lambda i: (0, 0)),
            pl.BlockSpec((1, 2 * d), lambda i: (0, 0)),
            pl.BlockSpec((1, fd), lambda i: (0, 0)),
        ],
        out_specs=[
            pl.BlockSpec((1, 1, _BE), lambda i: (i, 0, 0)),
            pl.BlockSpec((_BE, d), lambda i: (i, 0)),
        ],
        out_shape=[
            jax.ShapeDtypeStruct((nb, 1, _BE), jnp.float32),
            jax.ShapeDtypeStruct((e, d), jnp.float32),
        ],
    )(xg_dst, xg_src, attn3, wt, gamma2, beta2, bias2)
    return gate3, msg


def _fusion_body(*refs):
    x_ref = refs[0]
    agg_refs = refs[1:-4]
    wih_t_ref, whh_t_ref, bias_ref, out_ref = refs[-4:]
    agg = agg_refs[0][0] + agg_refs[0][1]
    for r in agg_refs[1:]:
        agg = agg + r[0] + r[1]
    out_ref[...] = (
        jnp.dot(jnp.maximum(agg, 0.0), wih_t_ref[...],
                preferred_element_type=jnp.float32)
        + jnp.dot(jnp.maximum(x_ref[...], 0.0), whh_t_ref[...],
                  preferred_element_type=jnp.float32)
        + bias_ref[...])


def _fusion(x, agg_parts, wih_t, whh_t, bias2):
    n, d = x.shape
    grid = (n // _BN,)
    return pl.pallas_call(
        _fusion_body,
        grid=grid,
        in_specs=[pl.BlockSpec((_BN, d), lambda i: (i, 0))]
        + [pl.BlockSpec((_NC, _BN, d), lambda i: (0, i, 0))
           for _ in agg_parts]
        + [
            pl.BlockSpec((d, d), lambda i: (0, 0)),
            pl.BlockSpec((d, d), lambda i: (0, 0)),
            pl.BlockSpec((1, d), lambda i: (0, 0)),
        ],
        out_specs=pl.BlockSpec((_BN, d), lambda i: (i, 0)),
        out_shape=jax.ShapeDtypeStruct((n, d), jnp.float32),
    )(x, *agg_parts, wih_t, whh_t, bias2)


_NCHUNK = 5


def kernel(x, edge_index, attn_value, ln_gamma, ln_beta, W, b,
           wih_W, wih_b, whh_W, whh_b):
    n, d = x.shape
    e = edge_index.shape[1]
    src = edge_index[0]
    dst = edge_index[1]

    ec = e // _NCHUNK
    n_pad = ((n + 8 * _NS - 1) // (8 * _NS)) * (8 * _NS)
    zeros = jnp.zeros((n_pad, d), jnp.float32)
    gather_fn = _make_gather(n, ec, d)
    scatter_fn = _make_scatter(n_pad, ec, d)
    wt = W.T.astype(jnp.bfloat16)
    gamma2 = ln_gamma.reshape(1, 2 * d)
    beta2 = ln_beta.reshape(1, 2 * d)
    b2 = b.reshape(1, -1)

    gates = []
    aggs = []
    for k in range(_NCHUNK):
        sl = slice(k * ec, (k + 1) * ec)
        xg_src, xg_dst = gather_fn(x, src[sl], dst[sl])
        attn3 = attn_value[sl].reshape(ec // _BE, 1, _BE)
        gate3, msg = _edge_compute(xg_dst, xg_src, attn3, wt,
                                   gamma2, beta2, b2)
        aggs.append(scatter_fn(msg, dst[sl].reshape(ec // _KS, 1, _KS),
                               zeros))
        gates.append(gate3.reshape(ec))

    out = _fusion(x, aggs, wih_W.T, whh_W.T, (wih_b + whh_b).reshape(1, d))
    return out, jnp.concatenate(gates)


# speedup vs baseline: 1.0693x; 1.0693x over previous
import functools

import jax
import jax.numpy as jnp
from jax import lax
from jax.experimental import pallas as pl
from jax.experimental.pallas import tpu as pltpu
from jax.experimental.pallas import tpu_sc as plsc

_NC = 2
_NS = 16
_KG = 80
_KS = 40
_BE = 2000
_BN = 2000


def _make_gather(n, e, d):
    nw = _NC * _NS
    epw = e // nw
    grp = 80
    ngrp = epw // grp
    nsub = grp // _KG
    stage_rows = (n // (8 * _NS)) * 8
    stage_rem = n - stage_rows * _NS
    mesh = plsc.VectorSubcoreMesh(core_axis_name="c", subcore_axis_name="s")

    @functools.partial(
        pl.kernel,
        mesh=mesh,
        out_type=[
            jax.ShapeDtypeStruct((e, d), jnp.float32),
            jax.ShapeDtypeStruct((e, d), jnp.float32),
        ],
        scratch_types=[
            pltpu.VMEM((epw,), jnp.int32),
            pltpu.VMEM((epw,), jnp.int32),
            pltpu.VMEM((grp, d), jnp.float32),
            pltpu.VMEM((grp, d), jnp.float32),
            pltpu.VMEM_SHARED((n, d), jnp.float32),
            pltpu.SemaphoreType.DMA,
            pltpu.SemaphoreType.DMA,
            pltpu.SemaphoreType.DMA,
        ],
    )
    def gather_k(x_hbm, src_hbm, dst_hbm, out_src, out_dst,
                 si_v, di_v, b0, b1, x_sh, gsem, ws0, ws1):
        wid = lax.axis_index("s") * _NC + lax.axis_index("c")
        s = lax.axis_index("s")
        base0 = wid * epw
        xr0 = s * stage_rows
        pltpu.sync_copy(x_hbm.at[pl.ds(xr0, stage_rows)],
                        x_sh.at[pl.ds(xr0, stage_rows)])
        if stage_rem:
            @pl.when(s == 0)
            def _():
                pltpu.sync_copy(
                    x_hbm.at[pl.ds(stage_rows * _NS, stage_rem)],
                    x_sh.at[pl.ds(stage_rows * _NS, stage_rem)])
        pltpu.sync_copy(src_hbm.at[pl.ds(base0, epw)], si_v)
        pltpu.sync_copy(dst_hbm.at[pl.ds(base0, epw)], di_v)
        plsc.subcore_barrier()

        def group(g, carry):
            o = g * grp

            @pl.when(g > 0)
            def _():
                pltpu.make_async_copy(
                    b0, out_src.at[pl.ds(base0, grp)], ws0).wait()

            cps = [pltpu.async_copy(
                x_sh.at[si_v.at[pl.ds(o + j * _KG, _KG)]],
                b0.at[pl.ds(j * _KG, _KG)], gsem) for j in range(nsub)]
            for cp in cps:
                cp.wait()
            pltpu.async_copy(b0, out_src.at[pl.ds(base0 + o, grp)], ws0)

            @pl.when(g > 0)
            def _():
                pltpu.make_async_copy(
                    b1, out_dst.at[pl.ds(base0, grp)], ws1).wait()

            cps = [pltpu.async_copy(
                x_sh.at[di_v.at[pl.ds(o + j * _KG, _KG)]],
                b1.at[pl.ds(j * _KG, _KG)], gsem) for j in range(nsub)]
            for cp in cps:
                cp.wait()
            pltpu.async_copy(b1, out_dst.at[pl.ds(base0 + o, grp)], ws1)
            return carry

        lax.fori_loop(0, ngrp, group, 0)
        pltpu.make_async_copy(b0, out_src.at[pl.ds(base0, grp)], ws0).wait()
        pltpu.make_async_copy(b1, out_dst.at[pl.ds(base0, grp)], ws1).wait()

    return gather_k


def _make_scatter(n_pad, e, d):
    epc = e // _NC
    ept = epc // _NS
    grp = _KS
    ngrp = ept // grp
    ring = 5
    niter = ngrp // ring
    nrow = ept // _KS
    rows_per_tile = n_pad // _NS
    mesh = plsc.VectorSubcoreMesh(core_axis_name="c", subcore_axis_name="s")

    @functools.partial(
        pl.kernel,
        mesh=mesh,
        out_type=jax.ShapeDtypeStruct((_NC, n_pad, d), jnp.float32),
        scratch_types=[
            pltpu.VMEM((nrow, 1, _KS), jnp.int32),
        ] + [pltpu.VMEM((grp, d), jnp.float32) for _ in range(ring)]
        + [pltpu.VMEM_SHARED((n_pad, d), jnp.float32)]
        + [pltpu.SemaphoreType.DMA for _ in range(ring)]
        + [pltpu.SemaphoreType.DMA],
    )
    def scatter_k(msg_hbm, dst2_hbm, zeros_hbm, out_hbm, idx2_v, *rest):
        m = rest[:ring]
        agg_sh = rest[ring]
        ls = rest[ring + 1:2 * ring + 1]
        asem = rest[2 * ring + 1]
        c = lax.axis_index("c")
        s = lax.axis_index("s")
        r0 = s * rows_per_tile
        pltpu.sync_copy(zeros_hbm.at[pl.ds(r0, rows_per_tile)],
                        agg_sh.at[pl.ds(r0, rows_per_tile)])
        base0 = c * epc + s * ept
        row0 = base0 // _KS
        pltpu.sync_copy(dst2_hbm.at[pl.ds(row0, nrow)], idx2_v)
        plsc.subcore_barrier()
        for j in range(ring - 1):
            pltpu.async_copy(msg_hbm.at[pl.ds(base0 + j * grp, grp)],
                             m[j], ls[j])

        def drain_one_add():
            pltpu.make_async_copy(
                m[0], agg_sh.at[idx2_v.at[0, 0]], asem).wait()

        def body(p, carry):
            for j in range(ring):
                g = p * ring + j
                pltpu.make_async_copy(
                    msg_hbm.at[pl.ds(base0, grp)], m[j], ls[j]).wait()
                pltpu.async_copy(m[j], agg_sh.at[idx2_v.at[g, 0]],
                                 asem, add=True)

                @pl.when(g + ring - 1 < ngrp)
                def _():
                    drain_one_add()
                    jf = (j + ring - 1) % ring
                    pltpu.async_copy(
                        msg_hbm.at[pl.ds(base0 + (g + ring - 1) * grp, grp)],
                        m[jf], ls[jf])
            return carry

        lax.fori_loop(0, niter, body, 0)
        for _ in range(ring - 1):
            drain_one_add()
        plsc.subcore_barrier()
        pltpu.sync_copy(agg_sh.at[pl.ds(r0, rows_per_tile)],
                        out_hbm.at[c, pl.ds(r0, rows_per_tile)])

    return scatter_k


def _edge_body(xd_ref, xs_ref, attn_ref, wt_ref, g_ref, bt_ref, bb_ref,
               gate_ref, msg_ref):
    xd = xd_ref[...]
    xs = xs_ref[...]
    d = xd.shape[1]
    two_d = 2.0 * d
    ssum = jnp.sum(xd, axis=1, keepdims=True) + jnp.sum(xs, axis=1, keepdims=True)
    ssq = (jnp.sum(xd * xd, axis=1, keepdims=True)
           + jnp.sum(xs * xs, axis=1, keepdims=True))
    mu = ssum / two_d
    var = ssq / two_d - mu * mu
    rstd = lax.rsqrt(var + 1e-5)
    gamma = g_ref[...]
    beta = bt_ref[...]
    ln_d = (xd - mu) * rstd * gamma[:, :d] + beta[:, :d]
    ln_s = (xs - mu) * rstd * gamma[:, d:] + beta[:, d:]
    rd = jnp.maximum(ln_d, 0.0)
    rs = jnp.maximum(ln_s, 0.0)
    wt = wt_ref[...]
    h = (jnp.dot(rd, wt[:d], preferred_element_type=jnp.float32)
         + jnp.dot(rs, wt[d:], preferred_element_type=jnp.float32)
         + bb_ref[...])
    gate = jnp.mean(jax.nn.sigmoid(h), axis=1)
    scale = gate * attn_ref[0, 0, :]
    msg_ref[...] = xs * scale[:, None]
    gate_ref[0, 0, :] = gate


def _edge_compute(xg_dst, xg_src, attn3, wt, gamma2, beta2, bias2):
    e, d = xg_src.shape
    nb = e // _BE
    fd = wt.shape[1]
    grid = (nb,)
    gate3, msg = pl.pallas_call(
        _edge_body,
        grid=grid,
        in_specs=[
            pl.BlockSpec((_BE, d), lambda i: (i, 0)),
            pl.BlockSpec((_BE, d), lambda i: (i, 0)),
            pl.BlockSpec((1, 1, _BE), lambda i: (i, 0, 0)),
            pl.BlockSpec((2 * d, fd), lambda i: (0, 0)),
            pl.BlockSpec((1, 2 * d), lambda i: (0, 0)),
            pl.BlockSpec((1, 2 * d), lambda i: (0, 0)),
            pl.BlockSpec((1, fd), lambda i: (0, 0)),
        ],
        out_specs=[
            pl.BlockSpec((1, 1, _BE), lambda i: (i, 0, 0)),
            pl.BlockSpec((_BE, d), lambda i: (i, 0)),
        ],
        out_shape=[
            jax.ShapeDtypeStruct((nb, 1, _BE), jnp.float32),
            jax.ShapeDtypeStruct((e, d), jnp.float32),
        ],
    )(xg_dst, xg_src, attn3, wt, gamma2, beta2, bias2)
    return gate3, msg


def _fusion_body(*refs):
    x_ref = refs[0]
    agg_refs = refs[1:-4]
    wih_t_ref, whh_t_ref, bias_ref, out_ref = refs[-4:]
    agg = agg_refs[0][0] + agg_refs[0][1]
    for r in agg_refs[1:]:
        agg = agg + r[0] + r[1]
    out_ref[...] = (
        jnp.dot(jnp.maximum(agg, 0.0), wih_t_ref[...],
                preferred_element_type=jnp.float32)
        + jnp.dot(jnp.maximum(x_ref[...], 0.0), whh_t_ref[...],
                  preferred_element_type=jnp.float32)
        + bias_ref[...])


def _fusion(x, agg_parts, wih_t, whh_t, bias2):
    n, d = x.shape
    grid = (n // _BN,)
    return pl.pallas_call(
        _fusion_body,
        grid=grid,
        in_specs=[pl.BlockSpec((_BN, d), lambda i: (i, 0))]
        + [pl.BlockSpec((_NC, _BN, d), lambda i: (0, i, 0))
           for _ in agg_parts]
        + [
            pl.BlockSpec((d, d), lambda i: (0, 0)),
            pl.BlockSpec((d, d), lambda i: (0, 0)),
            pl.BlockSpec((1, d), lambda i: (0, 0)),
        ],
        out_specs=pl.BlockSpec((_BN, d), lambda i: (i, 0)),
        out_shape=jax.ShapeDtypeStruct((n, d), jnp.float32),
    )(x, *agg_parts, wih_t, whh_t, bias2)


_NCHUNK = 5


def kernel(x, edge_index, attn_value, ln_gamma, ln_beta, W, b,
           wih_W, wih_b, whh_W, whh_b):
    n, d = x.shape
    e = edge_index.shape[1]
    src = edge_index[0]
    dst = edge_index[1]

    ec = e // _NCHUNK
    n_pad = ((n + 8 * _NS - 1) // (8 * _NS)) * (8 * _NS)
    zeros = jnp.zeros((n_pad, d), jnp.float32)
    gather_fn = _make_gather(n, ec, d)
    scatter_fn = _make_scatter(n_pad, ec, d)
    wt = W.T
    gamma2 = ln_gamma.reshape(1, 2 * d)
    beta2 = ln_beta.reshape(1, 2 * d)
    b2 = b.reshape(1, -1)

    gates = []
    aggs = []
    for k in range(_NCHUNK):
        sl = slice(k * ec, (k + 1) * ec)
        xg_src, xg_dst = gather_fn(x, src[sl], dst[sl])
        attn3 = attn_value[sl].reshape(ec // _BE, 1, _BE)
        gate3, msg = _edge_compute(xg_dst, xg_src, attn3, wt,
                                   gamma2, beta2, b2)
        aggs.append(scatter_fn(msg, dst[sl].reshape(ec // _KS, 1, _KS),
                               zeros))
        gates.append(gate3.reshape(ec))

    out = _fusion(x, aggs, wih_W.T, whh_W.T, (wih_b + whh_b).reshape(1, d))
    return out, jnp.concatenate(gates)
